# Initial kernel scaffold; baseline (speedup 1.0000x reference)
import jax
import jax.numpy as jnp
from jax.experimental import pallas as pl


def kernel(Xuser, Xap, batch_num, beta_open_loop):
    raise NotImplementedError("write your pallas kernel here")


# speedup vs baseline: 3.6022x; 3.6022x over previous
import functools

import jax
import jax.numpy as jnp
from jax.experimental import pallas as pl
from jax.experimental.pallas import tpu as pltpu

_NAP = 64
_NU = 640
_B2 = 64
_BN = 32
_EX = 100.0
_EY = 100.0
_P = -1.9
_C0 = -4.6 * 2.302585092994046


def _body(beta_ref, xu_rx, xu_ry, xu_cx, xu_cy, ap_rx, ap_ry, ap_cx, ap_cy,
          g_out, pp_out, sv_out):
    beta = beta_ref[0, 0]
    dx = jnp.abs(ap_cx[0] - xu_rx[0])
    dy = jnp.abs(ap_cy[0] - xu_ry[0])
    dxw = jnp.minimum(dx, _EX - dx)
    dyw = jnp.minimum(dy, _EY - dy)
    d2_au = dxw * dxw + dyw * dyw + 1.0

    tx = jnp.abs(xu_cx[0] - ap_rx[0])
    ty = jnp.abs(xu_cy[0] - ap_ry[0])
    txw = jnp.minimum(tx, _EX - tx)
    tyw = jnp.minimum(ty, _EY - ty)
    d2_ua = txw * txw + tyw * tyw + 1.0

    mn = jnp.min(d2_ua, axis=1, keepdims=True)
    ap_iota = jax.lax.broadcasted_iota(jnp.int32, (_NU, _NAP), 1)
    nearest = jnp.min(jnp.where(d2_ua == mn, ap_iota, _NAP), axis=1,
                      keepdims=True)

    u_iota = jax.lax.broadcasted_iota(jnp.int32, (_NU, _NAP), 0)
    maskT = nearest == ap_iota
    maskedT = jnp.where(maskT, u_iota, -1)
    sel_row = jnp.max(maskedT, axis=0, keepdims=True)
    onehot = jnp.where(jnp.logical_and(maskT, maskedT == sel_row),
                       1.0, 0.0).astype(jnp.float32)

    d_sel = jax.lax.dot_general(d2_au, onehot, (((1,), (0,)), ((), ())),
                                precision=jax.lax.Precision.HIGHEST,
                                preferred_element_type=jnp.float32)

    g = jnp.exp(_C0 + _P * jnp.log(d_sel))
    i_iota = jax.lax.broadcasted_iota(jnp.int32, (_NAP, _NAP), 0)
    j_iota = jax.lax.broadcasted_iota(jnp.int32, (_NAP, _NAP), 1)
    diag = jnp.sum(jnp.where(i_iota == j_iota, d_sel, 0.0), axis=0,
                   keepdims=True)
    pp = jnp.exp(-beta * (_C0 + _P * jnp.log(diag)))

    g_out[0] = g
    pp_out[0] = pp
    sv_out[0] = jnp.broadcast_to(jnp.min(sel_row, axis=1, keepdims=True),
                                 (1, _NAP))


def kernel(Xuser, Xap, batch_num, beta_open_loop):
    xu_x = Xuser[:, :, 0].astype(jnp.float32)
    xu_y = Xuser[:, :, 1].astype(jnp.float32)
    ap_x = Xap[:, :, 0].astype(jnp.float32)
    ap_y = Xap[:, :, 1].astype(jnp.float32)
    beta = jnp.asarray(beta_open_loop, jnp.float32).reshape(1, 1)

    args = (
        xu_x.reshape(_B2, 1, _NU), xu_y.reshape(_B2, 1, _NU),
        xu_x.reshape(_B2, _NU, 1), xu_y.reshape(_B2, _NU, 1),
        ap_x.reshape(_B2, 1, _NAP), ap_y.reshape(_B2, 1, _NAP),
        ap_x.reshape(_B2, _NAP, 1), ap_y.reshape(_B2, _NAP, 1),
    )
    specs = [
        pl.BlockSpec((1, 1, _NU), lambda b: (b, 0, 0)),
        pl.BlockSpec((1, 1, _NU), lambda b: (b, 0, 0)),
        pl.BlockSpec((1, _NU, 1), lambda b: (b, 0, 0)),
        pl.BlockSpec((1, _NU, 1), lambda b: (b, 0, 0)),
        pl.BlockSpec((1, 1, _NAP), lambda b: (b, 0, 0)),
        pl.BlockSpec((1, 1, _NAP), lambda b: (b, 0, 0)),
        pl.BlockSpec((1, _NAP, 1), lambda b: (b, 0, 0)),
        pl.BlockSpec((1, _NAP, 1), lambda b: (b, 0, 0)),
    ]
    g_full, pp_full, svmin = pl.pallas_call(
        _body,
        grid=(_B2,),
        in_specs=[pl.BlockSpec(memory_space=pltpu.SMEM)] + specs,
        out_specs=[
            pl.BlockSpec((1, _NAP, _NAP), lambda b: (b, 0, 0)),
            pl.BlockSpec((1, 1, _NAP), lambda b: (b, 0, 0)),
            pl.BlockSpec((1, 1, _NAP), lambda b: (b, 0, 0)),
        ],
        out_shape=[
            jax.ShapeDtypeStruct((_B2, _NAP, _NAP), jnp.float32),
            jax.ShapeDtypeStruct((_B2, 1, _NAP), jnp.float32),
            jax.ShapeDtypeStruct((_B2, 1, _NAP), jnp.int32),
        ],
    )(beta, *args)

    survive = svmin[:, 0, 0] >= 0
    order = jnp.argsort(jnp.logical_not(survive).astype(jnp.int32), stable=True)
    sel_b = order[:_BN] + (jnp.asarray(batch_num, jnp.int32) - _BN)
    G = jnp.take(g_full, sel_b, axis=0)
    power_propotional = jnp.take(pp_full[:, 0, :], sel_b, axis=0)
    return (G, power_propotional)
